# Initial kernel scaffold; baseline (speedup 1.0000x reference)
import jax
import jax.numpy as jnp
from jax.experimental import pallas as pl


def kernel(x, edge_index, W, b):
    raise NotImplementedError("write your pallas kernel here")


# speedup vs baseline: 15.3110x; 15.3110x over previous
import functools

import jax
import jax.numpy as jnp
from jax import lax
from jax.experimental import pallas as pl
from jax.experimental.pallas import tpu as pltpu
from jax.experimental.pallas import tpu_sc as plsc

N = 10000
E = 320000
D = 128
NC = 2
NS = 16
NW = NC * NS
NP = 10240
RPT = NP // NS
EPW = E // NW
C = 80
CHUNKS = EPW // C


def _sc_degree(row):
    mesh = plsc.VectorSubcoreMesh(core_axis_name="c", subcore_axis_name="s")

    @functools.partial(
        pl.kernel,
        mesh=mesh,
        out_type=jax.ShapeDtypeStruct((NC, NP), jnp.float32),
        scratch_types=[
            pltpu.VMEM((C,), jnp.int32),
            pltpu.VMEM((C,), jnp.float32),
            pltpu.VMEM((RPT,), jnp.float32),
            pltpu.VMEM_SHARED((NP,), jnp.float32),
        ],
    )
    def deg_kernel(row_hbm, out_hbm, idx_v, ones_v, zero_v, deg_sh):
        cid = lax.axis_index("c")
        sid = lax.axis_index("s")
        wid = sid * NC + cid

        def fill_z(i, carry):
            zero_v[pl.ds(i * 16, 16)] = jnp.zeros((16,), jnp.float32)
            return carry

        lax.fori_loop(0, RPT // 16, fill_z, 0)

        def fill_o(i, carry):
            ones_v[pl.ds(i * 16, 16)] = jnp.ones((16,), jnp.float32)
            return carry

        lax.fori_loop(0, C // 16, fill_o, 0)

        pltpu.sync_copy(zero_v, deg_sh.at[pl.ds(sid * RPT, RPT)])
        plsc.subcore_barrier()

        base0 = wid * EPW

        def body(j, carry):
            base = pl.multiple_of(base0 + j * C, 8)
            pltpu.sync_copy(row_hbm.at[pl.ds(base, C)], idx_v)
            pltpu.sync_copy(ones_v, deg_sh.at[idx_v], add=True)
            return carry

        lax.fori_loop(0, CHUNKS, body, 0)

        plsc.subcore_barrier()
        pltpu.sync_copy(deg_sh.at[pl.ds(sid * RPT, RPT)],
                        out_hbm.at[cid, pl.ds(sid * RPT, RPT)])

    return deg_kernel(row)


def _tc_scale(degT, x):
    def body(deg_ref, x_ref, xs_ref):
        d = deg_ref[:, 0:1] + deg_ref[:, 1:2]
        dc = lax.rsqrt(jnp.maximum(d, 1.0))[:N]
        xs_ref[...] = x_ref[...] * dc

    return pl.pallas_call(
        body,
        out_shape=jax.ShapeDtypeStruct((N, D), jnp.float32),
    )(degT, x)


def _sc_scatter(col, row, xs):
    mesh = plsc.VectorSubcoreMesh(core_axis_name="c", subcore_axis_name="s")

    @functools.partial(
        pl.kernel,
        mesh=mesh,
        out_type=jax.ShapeDtypeStruct((NC, NP, D), jnp.float32),
        scratch_types=[
            pltpu.VMEM((C,), jnp.int32),
            pltpu.VMEM((C,), jnp.int32),
            pltpu.VMEM((C, D), jnp.float32),
            pltpu.VMEM((C, D), jnp.float32),
            pltpu.VMEM_SHARED((NP, D), jnp.float32),
            pltpu.SemaphoreType.DMA,
        ],
    )
    def scat_kernel(col_hbm, row_hbm, xs_hbm, out_hbm,
                    colv, rowv, gbuf, zbuf, acc_sh, sem):
        cid = lax.axis_index("c")
        sid = lax.axis_index("s")
        wid = sid * NC + cid

        def fill_z(r, carry):
            for k in range(D // 16):
                zbuf[r, pl.ds(k * 16, 16)] = jnp.zeros((16,), jnp.float32)
            return carry

        lax.fori_loop(0, C, fill_z, 0)

        def zslice(z, carry):
            pltpu.sync_copy(zbuf, acc_sh.at[pl.ds(sid * RPT + z * C, C)])
            return carry

        lax.fori_loop(0, RPT // C, zslice, 0)
        plsc.subcore_barrier()

        base0 = wid * EPW

        def body(j, carry):
            base = pl.multiple_of(base0 + j * C, 8)
            pltpu.sync_copy(col_hbm.at[pl.ds(base, C)], colv)
            pltpu.sync_copy(row_hbm.at[pl.ds(base, C)], rowv)
            pltpu.async_copy(xs_hbm.at[colv], gbuf, sem).wait()
            pltpu.sync_copy(gbuf, acc_sh.at[rowv], add=True)
            return carry

        lax.fori_loop(0, CHUNKS, body, 0)

        plsc.subcore_barrier()

        def wout(z, carry):
            off = sid * RPT + z * C
            pltpu.sync_copy(acc_sh.at[pl.ds(off, C)],
                            out_hbm.at[cid, pl.ds(off, C)])
            return carry

        lax.fori_loop(0, RPT // C, wout, 0)

    return scat_kernel(col, row, xs)


def _tc_out(degT, acc_parts, W, b):
    def body(deg_ref, acc_ref, w_ref, b_ref, o_ref):
        d = deg_ref[:, 0:1] + deg_ref[:, 1:2]
        dr = lax.rsqrt(jnp.maximum(d, 1.0))[:N]
        acc = acc_ref[0, :N, :] + acc_ref[1, :N, :]
        a = acc * dr
        y = lax.dot_general(a, w_ref[...], (((1,), (1,)), ((), ())),
                            preferred_element_type=jnp.float32)
        o_ref[...] = y + b_ref[...][None, :]

    return pl.pallas_call(
        body,
        out_shape=jax.ShapeDtypeStruct((N, D), jnp.float32),
    )(degT, acc_parts, W, b)


def kernel(x, edge_index, W, b):
    row = edge_index[0]
    col = edge_index[1]
    deg_parts = _sc_degree(row)
    degT = deg_parts.T
    xs = _tc_scale(degT, x)
    acc_parts = _sc_scatter(col, row, xs)
    return _tc_out(degT, acc_parts, W, b)
